# baseline (speedup 1.0000x reference)
import jax
import jax.numpy as jnp
import numpy as np
from jax.experimental import pallas as pl
from jax.experimental.pallas import tpu as pltpu

_T = 16
_T_REMAIN = 12

_SEC = np.asarray(
    jnp.sort(jax.random.choice(jax.random.key(42), _T, shape=(_T_REMAIN,), replace=False))
)


def _runs_of(sec):
    runs = []
    i = 0
    while i < len(sec):
        j = i
        while j + 1 < len(sec) and sec[j + 1] == sec[j] + 1:
            j += 1
        runs.append((int(sec[i]), i, j - i + 1))
        i = j + 1
    return runs


_RUNS = _runs_of(_SEC)
_NSPLIT = 8


def kernel(x_seq):
    T, N, C, H, W = x_seq.shape
    row = N * C * H * W
    x_flat = x_seq.reshape(T * row)

    copies = []
    for src_row, dst_row, n_rows in _RUNS:
        ln = n_rows * row // _NSPLIT
        for j in range(_NSPLIT):
            copies.append((src_row * row + j * ln, dst_row * row + j * ln, ln))
    ncp = len(copies)

    def body(x_ref, o_ref, *sems):
        cs = []
        for k, (src, dst, ln) in enumerate(copies):
            c = pltpu.make_async_copy(
                x_ref.at[pl.ds(src, ln)], o_ref.at[pl.ds(dst, ln)], sems[k])
            c.start()
            cs.append(c)
        for c in cs:
            c.wait()

    out = pl.pallas_call(
        body,
        in_specs=[pl.BlockSpec(memory_space=pl.ANY)],
        out_specs=pl.BlockSpec(memory_space=pl.ANY),
        out_shape=jax.ShapeDtypeStruct((_T_REMAIN * row,), jnp.float32),
        scratch_shapes=[pltpu.SemaphoreType.DMA] * ncp,
    )(x_flat)

    return out.reshape(_T_REMAIN, N, C, H, W)


# speedup vs baseline: 33.9244x; 33.9244x over previous
import functools

import jax
import jax.numpy as jnp
import numpy as np
from jax import lax
from jax.experimental import pallas as pl
from jax.experimental.pallas import tpu as pltpu
from jax.experimental.pallas import tpu_sc as plsc

_T = 16
_T_REMAIN = 12

_SEC = np.asarray(
    jnp.sort(jax.random.choice(jax.random.key(42), _T, shape=(_T_REMAIN,), replace=False))
)


def _runs_of(sec):
    runs = []
    i = 0
    while i < len(sec):
        j = i
        while j + 1 < len(sec) and sec[j + 1] == sec[j] + 1:
            j += 1
        runs.append((int(sec[i]), i, j - i + 1))
        i = j + 1
    return runs


_RUNS = _runs_of(_SEC)

_info = plsc.get_sparse_core_info()
_NC, _NS = _info.num_cores, _info.num_subcores
_NW = _NC * _NS

_CH = 32768
_NBUF = 3
_LOOKAHEAD = 2


def _make_sc_gather(row_elems):
    mesh = plsc.VectorSubcoreMesh(core_axis_name="c", subcore_axis_name="s")
    out_elems = _T_REMAIN * row_elems

    chunks = []
    for src_row, dst_row, n_rows in _RUNS:
        plen = n_rows * row_elems // _NW
        assert plen % _CH == 0
        for j in range(plen // _CH):
            chunks.append((src_row * row_elems + j * _CH,
                           dst_row * row_elems + j * _CH,
                           plen))
    n = len(chunks)

    @functools.partial(
        pl.kernel,
        mesh=mesh,
        out_type=jax.ShapeDtypeStruct((out_elems,), jnp.float32),
        scratch_types=[pltpu.VMEM((_CH,), jnp.float32)] * _NBUF
        + [pltpu.SemaphoreType.DMA] * (2 * _NBUF),
    )
    def sc_gather(x_hbm, out_hbm, *scratch):
        bufs = scratch[:_NBUF]
        sin, sout = scratch[_NBUF:2 * _NBUF], scratch[2 * _NBUF:]
        wid = lax.axis_index("s") * _NC + lax.axis_index("c")

        def in_copy(j):
            src, _, plen = chunks[j]
            return pltpu.make_async_copy(
                x_hbm.at[pl.ds(src + wid * plen, _CH)],
                bufs[j % _NBUF], sin[j % _NBUF])

        def out_copy(j):
            _, dst, plen = chunks[j]
            return pltpu.make_async_copy(
                bufs[j % _NBUF],
                out_hbm.at[pl.ds(dst + wid * plen, _CH)], sout[j % _NBUF])

        for j in range(min(_LOOKAHEAD, n)):
            in_copy(j).start()
        for j in range(n):
            jj = j + _LOOKAHEAD
            if jj < n:
                if jj >= _NBUF:
                    out_copy(jj - _NBUF).wait()
                in_copy(jj).start()
            in_copy(j).wait()
            out_copy(j).start()
        for j in range(max(n - _NBUF, 0), n):
            out_copy(j).wait()

    return sc_gather


def kernel(x_seq):
    T, N, C, H, W = x_seq.shape
    row = N * C * H * W
    x_flat = x_seq.reshape(T * row)
    out = _make_sc_gather(row)(x_flat)
    return out.reshape(_T_REMAIN, N, C, H, W)
